# baseline (speedup 1.0000x reference)
import functools

import jax
import jax.numpy as jnp
from jax import lax
from jax.experimental import pallas as pl
from jax.experimental.pallas import tpu as pltpu
from jax.experimental.pallas import tpu_sc as plsc

B = 16384
D = 64

_NC, _NS = 2, 16
_NW = _NC * _NS
_BPW = B // _NW
_CH = 128
_NCH = _BPW // _CH
_L = 16


def _sc_gather_one(table_packed, ids):
    mesh = plsc.VectorSubcoreMesh(core_axis_name="c", subcore_axis_name="s")

    @functools.partial(
        pl.kernel,
        mesh=mesh,
        out_type=jax.ShapeDtypeStruct((B, 2 * D), jnp.float32),
        scratch_types=[
            pltpu.VMEM((_BPW,), jnp.int32),
            pltpu.VMEM((_BPW, 2 * D), jnp.float32),
            pltpu.SemaphoreType.DMA,
        ],
    )
    def k(t_hbm, id_hbm, out_hbm, idx_v, rows_v, sem):
        wid = lax.axis_index("s") * _NC + lax.axis_index("c")
        base = wid * _BPW
        pltpu.sync_copy(id_hbm.at[pl.ds(base, _BPW)], idx_v)
        copies = []
        for j in range(_NCH):
            sl = pl.ds(j * _CH, _CH)
            copies.append(
                pltpu.async_copy(t_hbm.at[idx_v.at[sl]], rows_v.at[sl], sem))
        for c in copies:
            c.wait()
        pltpu.sync_copy(rows_v, out_hbm.at[pl.ds(base, _BPW)])

    return k(table_packed, ids)


_BLK = 2048


def _mlp_body(ur_ref, mr_ref, upar_ref, mpar_ref, w1u_ref, w1m_ref, b1_ref,
              w2_ref, b2_ref, w3_ref, b3_ref, out_ref):
    ur = ur_ref[...]
    mr = mr_ref[...]
    upar = upar_ref[...]
    mpar = mpar_ref[...]
    u = ur[:, :D] * (1.0 - upar) + ur[:, D:] * upar
    m = mr[:, :D] * (1.0 - mpar) + mr[:, D:] * mpar
    h = (jnp.dot(u, w1u_ref[...], preferred_element_type=jnp.float32)
         + jnp.dot(m, w1m_ref[...], preferred_element_type=jnp.float32)
         + b1_ref[...])
    h = jnp.maximum(h, 0.0)
    h = jnp.dot(h, w2_ref[...], preferred_element_type=jnp.float32) + b2_ref[...]
    h = jnp.maximum(h, 0.0)
    out_ref[...] = (jnp.dot(h, w3_ref[...], preferred_element_type=jnp.float32)
                    + b3_ref[...])


def _tc_mlp(u_rows, m_rows, upar, mpar, W1, b1, W2, b2, W3, b3):
    out = pl.pallas_call(
        _mlp_body,
        grid=(B // _BLK,),
        in_specs=[
            pl.BlockSpec((_BLK, 2 * D), lambda i: (i, 0)),
            pl.BlockSpec((_BLK, 2 * D), lambda i: (i, 0)),
            pl.BlockSpec((_BLK, 1), lambda i: (i, 0)),
            pl.BlockSpec((_BLK, 1), lambda i: (i, 0)),
            pl.BlockSpec((D, 64), lambda i: (0, 0)),
            pl.BlockSpec((D, 64), lambda i: (0, 0)),
            pl.BlockSpec((1, 64), lambda i: (0, 0)),
            pl.BlockSpec((64, 32), lambda i: (0, 0)),
            pl.BlockSpec((1, 32), lambda i: (0, 0)),
            pl.BlockSpec((32, 1), lambda i: (0, 0)),
            pl.BlockSpec((1, 1), lambda i: (0, 0)),
        ],
        out_specs=pl.BlockSpec((_BLK, 1), lambda i: (i, 0)),
        out_shape=jax.ShapeDtypeStruct((B, 1), jnp.float32),
    )(u_rows, m_rows, upar, mpar, W1[:D], W1[D:], b1.reshape(1, 64), W2,
      b2.reshape(1, 32), W3, b3.reshape(1, 1))
    return out.reshape(B)


def _pack_pair_body(lb, x_ref, o_ref):
    yt = jnp.transpose(x_ref[...])
    for gq in range(lb // 128):
        o_ref[gq * 64:gq * 64 + 64, :D] = yt[gq * 128:gq * 128 + 64, :]
        o_ref[gq * 64:gq * 64 + 64, D:] = yt[gq * 128 + 64:gq * 128 + 128, :]


def _tc_pack(table_t, lb):
    n = table_t.shape[1]
    nrec = ((n + 127) // 128) * 64
    return pl.pallas_call(
        functools.partial(_pack_pair_body, lb),
        grid=(pl.cdiv(n, lb),),
        in_specs=[pl.BlockSpec((D, lb), lambda i: (0, i))],
        out_specs=pl.BlockSpec((lb // 2, 2 * D), lambda i: (i, 0)),
        out_shape=jax.ShapeDtypeStruct((nrec, 2 * D), jnp.float32),
    )(table_t)


def kernel(user_ids, movie_ids, user_table, movie_table, W1, b1, W2, b2, W3, b3):
    uids = user_ids.astype(jnp.int32)
    mids = movie_ids.astype(jnp.int32)
    m_packed = _tc_pack(movie_table.T, 16384)
    u_packed = _tc_pack(user_table.T, 16384)
    u_rows = _sc_gather_one(u_packed, (uids >> 7) * 64 + (uids & 63))
    m_rows = _sc_gather_one(m_packed, (mids >> 7) * 64 + (mids & 63))
    upar = ((uids >> 6) & 1).astype(jnp.float32).reshape(B, 1)
    mpar = ((mids >> 6) & 1).astype(jnp.float32).reshape(B, 1)
    return _tc_mlp(u_rows, m_rows, upar, mpar, W1, b1, W2, b2, W3, b3)


# speedup vs baseline: 1.0462x; 1.0462x over previous
import functools

import jax
import jax.numpy as jnp
from jax import lax
from jax.experimental import pallas as pl
from jax.experimental.pallas import tpu as pltpu
from jax.experimental.pallas import tpu_sc as plsc

B = 16384
D = 64

_NC, _NS = 2, 16
_NW = _NC * _NS
_BPW = B // _NW
_CH = 128
_NCH = _BPW // _CH
_L = 16


def _sc_gather_one(table_packed, ids):
    mesh = plsc.VectorSubcoreMesh(core_axis_name="c", subcore_axis_name="s")

    @functools.partial(
        pl.kernel,
        mesh=mesh,
        out_type=jax.ShapeDtypeStruct((B, 2 * D), jnp.float32),
        scratch_types=[
            pltpu.VMEM((_BPW,), jnp.int32),
            pltpu.VMEM((_BPW, 2 * D), jnp.float32),
            pltpu.SemaphoreType.DMA,
        ],
    )
    def k(t_hbm, id_hbm, out_hbm, idx_v, rows_v, sem):
        wid = lax.axis_index("s") * _NC + lax.axis_index("c")
        base = wid * _BPW
        pltpu.sync_copy(id_hbm.at[pl.ds(base, _BPW)], idx_v)
        copies = []
        for j in range(_NCH):
            sl = pl.ds(j * _CH, _CH)
            copies.append(
                pltpu.async_copy(t_hbm.at[idx_v.at[sl]], rows_v.at[sl], sem))
        for c in copies:
            c.wait()
        pltpu.sync_copy(rows_v, out_hbm.at[pl.ds(base, _BPW)])

    return k(table_packed, ids)


_BLK = 2048


def _mlp_body(ur_ref, mr_ref, upar_ref, mpar_ref, w1u_ref, w1m_ref, b1_ref,
              w2_ref, b2_ref, w3_ref, b3_ref, out_ref):
    ur = ur_ref[...]
    mr = mr_ref[...]
    upar = upar_ref[...]
    mpar = mpar_ref[...]
    u = ur[:, :D] * (1.0 - upar) + ur[:, D:] * upar
    m = mr[:, :D] * (1.0 - mpar) + mr[:, D:] * mpar
    h = (jnp.dot(u, w1u_ref[...], preferred_element_type=jnp.float32)
         + jnp.dot(m, w1m_ref[...], preferred_element_type=jnp.float32)
         + b1_ref[...])
    h = jnp.maximum(h, 0.0)
    h = jnp.dot(h, w2_ref[...], preferred_element_type=jnp.float32) + b2_ref[...]
    h = jnp.maximum(h, 0.0)
    out_ref[...] = (jnp.dot(h, w3_ref[...], preferred_element_type=jnp.float32)
                    + b3_ref[...])


def _tc_mlp(u_rows, m_rows, upar, mpar, W1, b1, W2, b2, W3, b3):
    out = pl.pallas_call(
        _mlp_body,
        grid=(B // _BLK,),
        in_specs=[
            pl.BlockSpec((_BLK, 2 * D), lambda i: (i, 0)),
            pl.BlockSpec((_BLK, 2 * D), lambda i: (i, 0)),
            pl.BlockSpec((_BLK, 1), lambda i: (i, 0)),
            pl.BlockSpec((_BLK, 1), lambda i: (i, 0)),
            pl.BlockSpec((D, 64), lambda i: (0, 0)),
            pl.BlockSpec((D, 64), lambda i: (0, 0)),
            pl.BlockSpec((1, 64), lambda i: (0, 0)),
            pl.BlockSpec((64, 32), lambda i: (0, 0)),
            pl.BlockSpec((1, 32), lambda i: (0, 0)),
            pl.BlockSpec((32, 1), lambda i: (0, 0)),
            pl.BlockSpec((1, 1), lambda i: (0, 0)),
        ],
        out_specs=pl.BlockSpec((_BLK, 1), lambda i: (i, 0)),
        out_shape=jax.ShapeDtypeStruct((B, 1), jnp.float32),
    )(u_rows, m_rows, upar, mpar, W1[:D], W1[D:], b1.reshape(1, 64), W2,
      b2.reshape(1, 32), W3, b3.reshape(1, 1))
    return out.reshape(B)


def _pack_pair_body(lb, x_ref, o_ref):
    yt = jnp.transpose(x_ref[...])
    for gq in range(lb // 128):
        o_ref[gq * 64:gq * 64 + 64, :D] = yt[gq * 128:gq * 128 + 64, :]
        o_ref[gq * 64:gq * 64 + 64, D:] = yt[gq * 128 + 64:gq * 128 + 128, :]


def _tc_pack(table_t, lb):
    n = table_t.shape[1]
    nrec = ((n + 127) // 128) * 64
    return pl.pallas_call(
        functools.partial(_pack_pair_body, lb),
        grid=(pl.cdiv(n, lb),),
        in_specs=[pl.BlockSpec((D, lb), lambda i: (0, i))],
        out_specs=pl.BlockSpec((lb // 2, 2 * D), lambda i: (i, 0)),
        out_shape=jax.ShapeDtypeStruct((nrec, 2 * D), jnp.float32),
    )(table_t)


def kernel(user_ids, movie_ids, user_table, movie_table, W1, b1, W2, b2, W3, b3):
    uids = user_ids.astype(jnp.int32)
    mids = movie_ids.astype(jnp.int32)
    m_packed = _tc_pack(movie_table.T, 16384)
    u_packed = _tc_pack(user_table.T, 32768)
    u_rows = _sc_gather_one(u_packed, (uids >> 7) * 64 + (uids & 63))
    m_rows = _sc_gather_one(m_packed, (mids >> 7) * 64 + (mids & 63))
    upar = ((uids >> 6) & 1).astype(jnp.float32).reshape(B, 1)
    mpar = ((mids >> 6) & 1).astype(jnp.float32).reshape(B, 1)
    return _tc_mlp(u_rows, m_rows, upar, mpar, W1, b1, W2, b2, W3, b3)
